# Initial kernel scaffold; baseline (speedup 1.0000x reference)
import jax
import jax.numpy as jnp
from jax.experimental import pallas as pl


def kernel(x, edge_index, hidden, W_src, W_dst, W_self, W_agg):
    raise NotImplementedError("write your pallas kernel here")


# speedup vs baseline: 3.8403x; 3.8403x over previous
import jax
import jax.numpy as jnp
from jax import lax
from jax.experimental import pallas as pl
from jax.experimental.pallas import tpu as pltpu
from jax.experimental.pallas import tpu_sc as plsc

N = 10000
D = 128
E = 320000
H = 128

NC = 2
NS = 16
NW = NC * NS
CH = 128
NCHUNK = 80
GRP = 8
EP = NW * NCHUNK * CH
NP = 10240
RPS = NP // NS
ZN = RPS // CH

RB = 1000
LANES = 16


def _proj_body(x_ref, h_ref, wx_ref, wh_ref, ps_ref, pd_ref, s_ref):
    o = (jnp.dot(x_ref[...], wx_ref[...], preferred_element_type=jnp.float32)
         + jnp.dot(h_ref[...], wh_ref[...], preferred_element_type=jnp.float32))
    ps_ref[...] = o[:, :H]
    pd_ref[...] = o[:, H:2 * H]
    s_ref[...] = o[:, 2 * H:]


def _proj(x, hidden, wx, wh):
    grid = N // RB
    return pl.pallas_call(
        _proj_body,
        grid=(grid,),
        in_specs=[
            pl.BlockSpec((RB, D), lambda i: (i, 0)),
            pl.BlockSpec((RB, H), lambda i: (i, 0)),
            pl.BlockSpec((D, 3 * H), lambda i: (0, 0)),
            pl.BlockSpec((H, 3 * H), lambda i: (0, 0)),
        ],
        out_specs=[
            pl.BlockSpec((RB, H), lambda i: (i, 0)),
            pl.BlockSpec((RB, H), lambda i: (i, 0)),
            pl.BlockSpec((RB, H), lambda i: (i, 0)),
        ],
        out_shape=[jax.ShapeDtypeStruct((N, H), jnp.float32)] * 3,
    )(x, hidden, wx, wh)


def _edge_body(ps_hbm, pd_hbm, src_hbm, dst_hbm, out_hbm,
               src_v, dst_v, rows_a, rows_b, agg_sh, sem_a, sem_b):
    c = lax.axis_index("c")
    s = lax.axis_index("s")
    w = c * NS + s
    base = s * RPS

    def _zrow(r, carry):
        def _zcol(k, carry2):
            rows_a[r, pl.ds(k * LANES, LANES)] = jnp.zeros((LANES,), jnp.float32)
            return carry2
        return lax.fori_loop(0, H // LANES, _zcol, carry)
    lax.fori_loop(0, CH, _zrow, 0)

    def _zslice(t, carry):
        pltpu.sync_copy(rows_a, agg_sh.at[pl.ds(base + t * CH, CH)])
        return carry
    lax.fori_loop(0, ZN, _zslice, 0)
    plsc.subcore_barrier()

    def _group(g, carry):
        pltpu.sync_copy(src_hbm.at[w, pl.ds(g * GRP, GRP)], src_v)
        pltpu.sync_copy(dst_hbm.at[w, pl.ds(g * GRP, GRP)], dst_v)

        def _chunk(j, carry1):
            ca = pltpu.async_copy(ps_hbm.at[src_v.at[j]], rows_a, sem_a)
            cb = pltpu.async_copy(pd_hbm.at[dst_v.at[j]], rows_b, sem_b)
            ca.wait()
            cb.wait()

            def _crow(r, carry2):
                def _ccol(k, carry3):
                    sl = pl.ds(k * LANES, LANES)
                    rows_a[r, sl] = jnp.maximum(rows_a[r, sl] + rows_b[r, sl], 0.0)
                    return carry3
                return lax.fori_loop(0, H // LANES, _ccol, carry2)
            lax.fori_loop(0, CH, _crow, 0)

            pltpu.sync_copy(rows_a, agg_sh.at[dst_v.at[j]], add=True)
            return carry1
        lax.fori_loop(0, GRP, _chunk, 0)
        return carry
    lax.fori_loop(0, NCHUNK // GRP, _group, 0)

    plsc.subcore_barrier()

    def _wslice(t, carry):
        off = base + t * CH
        pltpu.sync_copy(agg_sh.at[pl.ds(off, CH)], rows_a)
        pltpu.sync_copy(rows_a, out_hbm.at[c, pl.ds(off, CH)])
        return carry
    lax.fori_loop(0, ZN, _wslice, 0)


def _edge(ps, pd, src3, dst3):
    mesh = plsc.VectorSubcoreMesh(core_axis_name="c", subcore_axis_name="s")
    f = pl.kernel(
        _edge_body,
        out_type=jax.ShapeDtypeStruct((NC, NP, H), jnp.float32),
        mesh=mesh,
        scratch_types=[
            pltpu.VMEM((GRP, CH), jnp.int32),
            pltpu.VMEM((GRP, CH), jnp.int32),
            pltpu.VMEM((CH, H), jnp.float32),
            pltpu.VMEM((CH, H), jnp.float32),
            pltpu.VMEM_SHARED((NP, H), jnp.float32),
            pltpu.SemaphoreType.DMA,
            pltpu.SemaphoreType.DMA,
        ],
    )
    return f(ps, pd, src3, dst3)


def _final_body(s_ref, agg_ref, w_ref, o_ref):
    a = agg_ref[0] + agg_ref[1]
    o_ref[...] = jnp.maximum(
        s_ref[...] + jnp.dot(a, w_ref[...], preferred_element_type=jnp.float32),
        0.0)


def _final(s, agg2, w_agg):
    grid = N // RB
    return pl.pallas_call(
        _final_body,
        grid=(grid,),
        in_specs=[
            pl.BlockSpec((RB, H), lambda i: (i, 0)),
            pl.BlockSpec((NC, RB, H), lambda i: (0, i, 0)),
            pl.BlockSpec((H, H), lambda i: (0, 0)),
        ],
        out_specs=pl.BlockSpec((RB, H), lambda i: (i, 0)),
        out_shape=jax.ShapeDtypeStruct((N, H), jnp.float32),
    )(s, agg2, w_agg)


def kernel(x, edge_index, hidden, W_src, W_dst, W_self, W_agg):
    ei = edge_index.astype(jnp.int32)
    npad = EP - E
    src = jnp.concatenate([ei[0], jnp.zeros((npad,), jnp.int32)])
    dst = jnp.concatenate([ei[1], jnp.full((npad,), N, jnp.int32)])
    src3 = src.reshape(NW, NCHUNK, CH)
    dst3 = dst.reshape(NW, NCHUNK, CH)
    wx = jnp.concatenate([W_src[:D], W_dst[:D], W_self[:D]], axis=1)
    wh = jnp.concatenate([W_src[D:], W_dst[D:], W_self[D:]], axis=1)
    ps, pd, s = _proj(x, hidden, wx, wh)
    agg2 = _edge(ps, pd, src3, dst3)
    return _final(s, agg2, W_agg)


# speedup vs baseline: 4.6915x; 1.2217x over previous
import jax
import jax.numpy as jnp
from jax import lax
from jax.experimental import pallas as pl
from jax.experimental.pallas import tpu as pltpu
from jax.experimental.pallas import tpu_sc as plsc

N = 10000
D = 128
E = 320000
H = 128

NC = 2
NS = 16
NW = NC * NS
CH = 64
NCHUNK = 160
EPW = NCHUNK * CH
EP = NW * EPW
NP = 10048
NT = NP // CH

RB = 1000
LANES = 16


def _proj_body(x_ref, h_ref, wx_ref, wh_ref, ps_ref, pd_ref, s_ref):
    o = (jnp.dot(x_ref[...], wx_ref[...], preferred_element_type=jnp.float32)
         + jnp.dot(h_ref[...], wh_ref[...], preferred_element_type=jnp.float32))
    ps_ref[...] = o[:, :H]
    pd_ref[...] = o[:, H:2 * H]
    s_ref[...] = o[:, 2 * H:]


def _proj(x, hidden, wx, wh):
    grid = N // RB
    return pl.pallas_call(
        _proj_body,
        grid=(grid,),
        in_specs=[
            pl.BlockSpec((RB, D), lambda i: (i, 0)),
            pl.BlockSpec((RB, H), lambda i: (i, 0)),
            pl.BlockSpec((D, 3 * H), lambda i: (0, 0)),
            pl.BlockSpec((H, 3 * H), lambda i: (0, 0)),
        ],
        out_specs=[
            pl.BlockSpec((RB, H), lambda i: (i, 0)),
            pl.BlockSpec((RB, H), lambda i: (i, 0)),
            pl.BlockSpec((RB, H), lambda i: (i, 0)),
        ],
        out_shape=[jax.ShapeDtypeStruct((N, H), jnp.float32)] * 3,
    )(x, hidden, wx, wh)


def _edge_body(ps_hbm, pd_hbm, src_hbm, dst_hbm, out_hbm,
               a0, a1, b0, b1, m0, m1,
               is0, is1, id0, id1, d20, d21, agg_sh,
               gsem0, gsem1, isem0, isem1, ssem0, ssem1):
    c = lax.axis_index("c")
    s = lax.axis_index("s")
    w = c * NS + s
    ebase = w * EPW

    a = (a0, a1)
    b = (b0, b1)
    m = (m0, m1)
    isv = (is0, is1)
    idv = (id0, id1)
    d2 = (d20, d21)
    gsem = (gsem0, gsem1)
    isem = (isem0, isem1)
    ssem = (ssem0, ssem1)

    def _zrow(r, carry):
        for k in range(H // LANES):
            m0[r, pl.ds(k * LANES, LANES)] = jnp.zeros((LANES,), jnp.float32)
        return carry
    lax.fori_loop(0, CH, _zrow, 0)

    ntiles = (NT - s + NS - 1) // NS

    def _ztile(t, carry):
        pltpu.sync_copy(m0, agg_sh.at[pl.ds((s + t * NS) * CH, CH)])
        return carry
    lax.fori_loop(0, ntiles, _ztile, 0)
    plsc.subcore_barrier()

    def issue_idx(j, p):
        pltpu.async_copy(src_hbm.at[pl.ds(ebase + j * CH, CH)], isv[p], isem[p])
        pltpu.async_copy(dst_hbm.at[pl.ds(ebase + j * CH, CH)], idv[p], isem[p])

    def wait_idx(p):
        pltpu.make_async_copy(src_hbm.at[pl.ds(0, CH)], isv[p], isem[p]).wait()
        pltpu.make_async_copy(dst_hbm.at[pl.ds(0, CH)], idv[p], isem[p]).wait()

    def issue_gather(p):
        pltpu.async_copy(ps_hbm.at[isv[p]], a[p], gsem[p])
        pltpu.async_copy(pd_hbm.at[idv[p]], b[p], gsem[p])

    def wait_gather(p):
        pltpu.make_async_copy(ps_hbm.at[isv[p]], a[p], gsem[p]).wait()
        pltpu.make_async_copy(pd_hbm.at[idv[p]], b[p], gsem[p]).wait()

    def issue_scatter(p):
        pltpu.async_copy(m[p], agg_sh.at[d2[p]], ssem[p], add=True)

    def wait_scatter(p):
        pltpu.make_async_copy(m[p], agg_sh.at[d2[p]], ssem[p]).wait()

    def save_idx(p):
        for k in range(CH // LANES):
            sl = pl.ds(k * LANES, LANES)
            d2[p][sl] = idv[p][sl]

    def compute(p):
        ap, bp, mp = a[p], b[p], m[p]

        def _crow(r, carry):
            for k in range(H // LANES):
                sl = pl.ds(k * LANES, LANES)
                mp[r, sl] = jnp.maximum(ap[r, sl] + bp[r, sl], 0.0)
            return carry
        lax.fori_loop(0, CH, _crow, 0)

    issue_idx(0, 0)
    issue_idx(1, 1)
    wait_idx(0)
    issue_gather(0)

    def _macro(t, carry):
        @pl.when(t >= 1)
        def _():
            wait_scatter(0)
        wait_idx(1)
        issue_gather(1)
        wait_gather(0)
        save_idx(0)

        @pl.when(t < NCHUNK // 2 - 1)
        def _():
            issue_idx(2 * t + 2, 0)
        compute(0)
        issue_scatter(0)

        @pl.when(t >= 1)
        def _():
            wait_scatter(1)

        @pl.when(t < NCHUNK // 2 - 1)
        def _():
            wait_idx(0)
            issue_gather(0)
        wait_gather(1)
        save_idx(1)

        @pl.when(t < NCHUNK // 2 - 1)
        def _():
            issue_idx(2 * t + 3, 1)
        compute(1)
        issue_scatter(1)
        return carry
    lax.fori_loop(0, NCHUNK // 2, _macro, 0)

    wait_scatter(0)
    wait_scatter(1)
    plsc.subcore_barrier()

    def _wtile(t, carry):
        off = (s + t * NS) * CH
        pltpu.sync_copy(agg_sh.at[pl.ds(off, CH)], m0)
        pltpu.sync_copy(m0, out_hbm.at[c, pl.ds(off, CH)])
        return carry
    lax.fori_loop(0, ntiles, _wtile, 0)


def _edge(ps, pd, src, dst):
    mesh = plsc.VectorSubcoreMesh(core_axis_name="c", subcore_axis_name="s")
    f = pl.kernel(
        _edge_body,
        out_type=jax.ShapeDtypeStruct((NC, NP, H), jnp.float32),
        mesh=mesh,
        scratch_types=[
            pltpu.VMEM((CH, H), jnp.float32),
            pltpu.VMEM((CH, H), jnp.float32),
            pltpu.VMEM((CH, H), jnp.float32),
            pltpu.VMEM((CH, H), jnp.float32),
            pltpu.VMEM((CH, H), jnp.float32),
            pltpu.VMEM((CH, H), jnp.float32),
            pltpu.VMEM((CH,), jnp.int32),
            pltpu.VMEM((CH,), jnp.int32),
            pltpu.VMEM((CH,), jnp.int32),
            pltpu.VMEM((CH,), jnp.int32),
            pltpu.VMEM((CH,), jnp.int32),
            pltpu.VMEM((CH,), jnp.int32),
            pltpu.VMEM_SHARED((NP, H), jnp.float32),
            pltpu.SemaphoreType.DMA,
            pltpu.SemaphoreType.DMA,
            pltpu.SemaphoreType.DMA,
            pltpu.SemaphoreType.DMA,
            pltpu.SemaphoreType.DMA,
            pltpu.SemaphoreType.DMA,
        ],
    )
    return f(ps, pd, src, dst)


def _final_body(s_ref, agg_ref, w_ref, o_ref):
    a = agg_ref[0] + agg_ref[1]
    o_ref[...] = jnp.maximum(
        s_ref[...] + jnp.dot(a, w_ref[...], preferred_element_type=jnp.float32),
        0.0)


def _final(s, agg2, w_agg):
    grid = N // RB
    return pl.pallas_call(
        _final_body,
        grid=(grid,),
        in_specs=[
            pl.BlockSpec((RB, H), lambda i: (i, 0)),
            pl.BlockSpec((NC, RB, H), lambda i: (0, i, 0)),
            pl.BlockSpec((H, H), lambda i: (0, 0)),
        ],
        out_specs=pl.BlockSpec((RB, H), lambda i: (i, 0)),
        out_shape=jax.ShapeDtypeStruct((N, H), jnp.float32),
    )(s, agg2, w_agg)


def kernel(x, edge_index, hidden, W_src, W_dst, W_self, W_agg):
    ei = edge_index.astype(jnp.int32)
    npad = EP - E
    src = jnp.concatenate([ei[0], jnp.zeros((npad,), jnp.int32)])
    dst = jnp.concatenate([ei[1], jnp.full((npad,), N, jnp.int32)])
    wx = jnp.concatenate([W_src[:D], W_dst[:D], W_self[:D]], axis=1)
    wh = jnp.concatenate([W_src[D:], W_dst[D:], W_self[D:]], axis=1)
    ps, pd, s = _proj(x, hidden, wx, wh)
    agg2 = _edge(ps, pd, src, dst)
    return _final(s, agg2, W_agg)


# speedup vs baseline: 5.3093x; 1.1317x over previous
import jax
import jax.numpy as jnp
from jax import lax
from jax.experimental import pallas as pl
from jax.experimental.pallas import tpu as pltpu
from jax.experimental.pallas import tpu_sc as plsc

N = 10000
D = 128
E = 320000
H = 128

NC = 2
NS = 16
NW = NC * NS
CH = 64
NCHUNK = 160
EPW = NCHUNK * CH
EP = NW * EPW
NP = 10048
NT = NP // CH

RB = 1000
LANES = 16


def _proj_body(x_ref, h_ref, wx_ref, wh_ref, ps_ref, pd_ref, s_ref):
    o = (jnp.dot(x_ref[...], wx_ref[...], preferred_element_type=jnp.float32)
         + jnp.dot(h_ref[...], wh_ref[...], preferred_element_type=jnp.float32))
    ps_ref[...] = o[:, :H]
    pd_ref[...] = o[:, H:2 * H]
    s_ref[...] = o[:, 2 * H:]


def _proj(x, hidden, wx, wh):
    grid = N // RB
    return pl.pallas_call(
        _proj_body,
        grid=(grid,),
        in_specs=[
            pl.BlockSpec((RB, D), lambda i: (i, 0)),
            pl.BlockSpec((RB, H), lambda i: (i, 0)),
            pl.BlockSpec((D, 3 * H), lambda i: (0, 0)),
            pl.BlockSpec((H, 3 * H), lambda i: (0, 0)),
        ],
        out_specs=[
            pl.BlockSpec((RB, H), lambda i: (i, 0)),
            pl.BlockSpec((RB, H), lambda i: (i, 0)),
            pl.BlockSpec((RB, H), lambda i: (i, 0)),
        ],
        out_shape=[jax.ShapeDtypeStruct((N, H), jnp.float32)] * 3,
    )(x, hidden, wx, wh)


def _edge_body(ps_hbm, pd_hbm, src_hbm, dst_hbm, out_hbm,
               a0, a1, b0, b1, m0, m1,
               is0, is1, id0, id1, d20, d21, agg_sh,
               gsem0, gsem1, isem0, isem1, ssem0, ssem1):
    c = lax.axis_index("c")
    s = lax.axis_index("s")
    w = c * NS + s
    ebase = w * EPW

    a = (a0, a1)
    b = (b0, b1)
    m = (m0, m1)
    isv = (is0, is1)
    idv = (id0, id1)
    d2 = (d20, d21)
    gsem = (gsem0, gsem1)
    isem = (isem0, isem1)
    ssem = (ssem0, ssem1)

    def _zrow(r, carry):
        for k in range(H // LANES):
            m0[r, pl.ds(k * LANES, LANES)] = jnp.zeros((LANES,), jnp.float32)
        return carry
    lax.fori_loop(0, CH, _zrow, 0)

    ntiles = (NT - s + NS - 1) // NS

    def _ztile(t, carry):
        pltpu.sync_copy(m0, agg_sh.at[pl.ds((s + t * NS) * CH, CH)])
        return carry
    lax.fori_loop(0, ntiles, _ztile, 0)
    plsc.subcore_barrier()

    def issue_idx(j, p):
        pltpu.async_copy(src_hbm.at[pl.ds(ebase + j * CH, CH)], isv[p], isem[p])
        pltpu.async_copy(dst_hbm.at[pl.ds(ebase + j * CH, CH)], idv[p], isem[p])

    def wait_idx(p):
        pltpu.make_async_copy(src_hbm.at[pl.ds(0, CH)], isv[p], isem[p]).wait()
        pltpu.make_async_copy(dst_hbm.at[pl.ds(0, CH)], idv[p], isem[p]).wait()

    def issue_gather(p):
        pltpu.async_copy(ps_hbm.at[isv[p]], a[p], gsem[p])
        pltpu.async_copy(pd_hbm.at[idv[p]], b[p], gsem[p])

    def wait_gather(p):
        pltpu.make_async_copy(ps_hbm.at[isv[p]], a[p], gsem[p]).wait()
        pltpu.make_async_copy(pd_hbm.at[idv[p]], b[p], gsem[p]).wait()

    def issue_scatter(p):
        pltpu.async_copy(m[p], agg_sh.at[d2[p]], ssem[p], add=True)

    def wait_scatter(p):
        pltpu.make_async_copy(m[p], agg_sh.at[d2[p]], ssem[p]).wait()

    def save_idx(p):
        for k in range(CH // LANES):
            sl = pl.ds(k * LANES, LANES)
            d2[p][sl] = idv[p][sl]

    def compute(p):
        ap, bp, mp = a[p], b[p], m[p]

        def _crow(r, carry):
            for k in range(H // LANES):
                sl = pl.ds(k * LANES, LANES)
                mp[r, sl] = jnp.maximum(ap[r, sl] + bp[r, sl], 0.0)
            return carry
        lax.fori_loop(0, CH, _crow, 0)

    issue_idx(0, 0)
    issue_idx(1, 1)
    wait_idx(0)
    issue_gather(0)

    def _macro(t, carry):
        @pl.when(t >= 1)
        def _():
            wait_scatter(0)
        wait_idx(1)
        issue_gather(1)
        wait_gather(0)
        save_idx(0)

        @pl.when(t < NCHUNK // 2 - 1)
        def _():
            issue_idx(2 * t + 2, 0)
        compute(0)
        issue_scatter(0)

        @pl.when(t >= 1)
        def _():
            wait_scatter(1)

        @pl.when(t < NCHUNK // 2 - 1)
        def _():
            wait_idx(0)
            issue_gather(0)
        wait_gather(1)
        save_idx(1)

        @pl.when(t < NCHUNK // 2 - 1)
        def _():
            issue_idx(2 * t + 3, 1)
        compute(1)
        issue_scatter(1)
        return carry
    lax.fori_loop(0, NCHUNK // 2, _macro, 0)

    wait_scatter(0)
    wait_scatter(1)
    plsc.subcore_barrier()

    def _wtile(t, carry):
        off = (s + t * NS) * CH
        pltpu.sync_copy(agg_sh.at[pl.ds(off, CH)], m0)
        pltpu.sync_copy(m0, out_hbm.at[c, pl.ds(off, CH)])
        return carry
    lax.fori_loop(0, ntiles, _wtile, 0)


def _edge(ps, pd, src, dst):
    mesh = plsc.VectorSubcoreMesh(core_axis_name="c", subcore_axis_name="s")
    f = pl.kernel(
        _edge_body,
        out_type=jax.ShapeDtypeStruct((NC, NP, H), jnp.float32),
        mesh=mesh,
        scratch_types=[
            pltpu.VMEM((CH, H), jnp.float32),
            pltpu.VMEM((CH, H), jnp.float32),
            pltpu.VMEM((CH, H), jnp.float32),
            pltpu.VMEM((CH, H), jnp.float32),
            pltpu.VMEM((CH, H), jnp.float32),
            pltpu.VMEM((CH, H), jnp.float32),
            pltpu.VMEM((CH,), jnp.int32),
            pltpu.VMEM((CH,), jnp.int32),
            pltpu.VMEM((CH,), jnp.int32),
            pltpu.VMEM((CH,), jnp.int32),
            pltpu.VMEM((CH,), jnp.int32),
            pltpu.VMEM((CH,), jnp.int32),
            pltpu.VMEM_SHARED((NP, H), jnp.float32),
            pltpu.SemaphoreType.DMA,
            pltpu.SemaphoreType.DMA,
            pltpu.SemaphoreType.DMA,
            pltpu.SemaphoreType.DMA,
            pltpu.SemaphoreType.DMA,
            pltpu.SemaphoreType.DMA,
        ],
    )
    return f(ps, pd, src, dst)


def _final_body(s_ref, agg_ref, w_ref, o_ref):
    a = agg_ref[0] + agg_ref[1]
    o_ref[...] = jnp.maximum(
        s_ref[...] + jnp.dot(a, w_ref[...], preferred_element_type=jnp.float32),
        0.0)


def _final(s, agg2, w_agg):
    grid = N // RB
    return pl.pallas_call(
        _final_body,
        grid=(grid,),
        in_specs=[
            pl.BlockSpec((RB, H), lambda i: (i, 0)),
            pl.BlockSpec((NC, RB, H), lambda i: (0, i, 0)),
            pl.BlockSpec((H, H), lambda i: (0, 0)),
        ],
        out_specs=pl.BlockSpec((RB, H), lambda i: (i, 0)),
        out_shape=jax.ShapeDtypeStruct((N, H), jnp.float32),
    )(s, agg2, w_agg)


def kernel(x, edge_index, hidden, W_src, W_dst, W_self, W_agg):
    ei = edge_index.astype(jnp.int32)
    npad = EP - E
    src = jnp.concatenate([ei[0], jnp.zeros((npad,), jnp.int32)])
    dst = jnp.concatenate(
        [ei[1], N + (jnp.arange(npad, dtype=jnp.int32) % (NP - N))])
    wx = jnp.concatenate([W_src[:D], W_dst[:D], W_self[:D]], axis=1)
    wh = jnp.concatenate([W_src[D:], W_dst[D:], W_self[D:]], axis=1)
    ps, pd, s = _proj(x, hidden, wx, wh)
    agg2 = _edge(ps, pd, src, dst)
    return _final(s, agg2, W_agg)


# speedup vs baseline: 5.7961x; 1.0917x over previous
import jax
import jax.numpy as jnp
from jax import lax
from jax.experimental import pallas as pl
from jax.experimental.pallas import tpu as pltpu
from jax.experimental.pallas import tpu_sc as plsc

N = 10000
D = 128
E = 320000
H = 128

NC = 2
NS = 16
NW = NC * NS
CH = 64
NCHUNK = 160
EPW = NCHUNK * CH
EP = NW * EPW
NP = 10048
NT = NP // CH

RB = 1000
LANES = 16


def _proj_body(x_ref, h_ref, wx_ref, wh_ref, ps_ref, pd_ref, s_ref):
    o = (jnp.dot(x_ref[...], wx_ref[...], preferred_element_type=jnp.float32)
         + jnp.dot(h_ref[...], wh_ref[...], preferred_element_type=jnp.float32))
    ps_ref[...] = o[:, :H]
    pd_ref[...] = o[:, H:2 * H]
    s_ref[...] = o[:, 2 * H:]


def _proj(x, hidden, wx, wh):
    grid = N // RB
    return pl.pallas_call(
        _proj_body,
        grid=(grid,),
        in_specs=[
            pl.BlockSpec((RB, D), lambda i: (i, 0)),
            pl.BlockSpec((RB, H), lambda i: (i, 0)),
            pl.BlockSpec((D, 3 * H), lambda i: (0, 0)),
            pl.BlockSpec((H, 3 * H), lambda i: (0, 0)),
        ],
        out_specs=[
            pl.BlockSpec((RB, H), lambda i: (i, 0)),
            pl.BlockSpec((RB, H), lambda i: (i, 0)),
            pl.BlockSpec((RB, H), lambda i: (i, 0)),
        ],
        out_shape=[jax.ShapeDtypeStruct((N, H), jnp.float32)] * 3,
    )(x, hidden, wx, wh)


def _edge_body(ps_hbm, pd_hbm, src_hbm, dst_hbm, out_hbm,
               a0, a1, b0, b1, m0, m1,
               is0, is1, id0, id1, d20, d21, agg_sh,
               gsem0, gsem1, isem0, isem1, ssem0, ssem1):
    c = lax.axis_index("c")
    s = lax.axis_index("s")
    w = c * NS + s
    ebase = w * EPW

    a = (a0, a1)
    b = (b0, b1)
    m = (m0, m1)
    isv = (is0, is1)
    idv = (id0, id1)
    d2 = (d20, d21)
    gsem = (gsem0, gsem1)
    isem = (isem0, isem1)
    ssem = (ssem0, ssem1)

    def _zrow(r, carry):
        for k in range(H // LANES):
            m0[r, pl.ds(k * LANES, LANES)] = jnp.zeros((LANES,), jnp.float32)
        return carry
    lax.fori_loop(0, CH, _zrow, 0)

    ntiles = (NT - s + NS - 1) // NS

    def _ztile(t, carry):
        pltpu.sync_copy(m0, agg_sh.at[pl.ds((s + t * NS) * CH, CH)])
        return carry
    lax.fori_loop(0, ntiles, _ztile, 0)
    plsc.subcore_barrier()

    def issue_idx(j, p):
        pltpu.async_copy(src_hbm.at[pl.ds(ebase + j * CH, CH)], isv[p], isem[p])
        pltpu.async_copy(dst_hbm.at[pl.ds(ebase + j * CH, CH)], idv[p], isem[p])

    def wait_idx(p):
        pltpu.make_async_copy(src_hbm.at[pl.ds(0, CH)], isv[p], isem[p]).wait()
        pltpu.make_async_copy(dst_hbm.at[pl.ds(0, CH)], idv[p], isem[p]).wait()

    def issue_gather(p):
        pltpu.async_copy(ps_hbm.at[isv[p]], a[p], gsem[p])
        pltpu.async_copy(pd_hbm.at[idv[p]], b[p], gsem[p])

    def wait_gather(p):
        pltpu.make_async_copy(ps_hbm.at[isv[p]], a[p], gsem[p]).wait()
        pltpu.make_async_copy(pd_hbm.at[idv[p]], b[p], gsem[p]).wait()

    def issue_scatter(p):
        pltpu.async_copy(m[p], agg_sh.at[d2[p]], ssem[p], add=True)

    def wait_scatter(p):
        pltpu.make_async_copy(m[p], agg_sh.at[d2[p]], ssem[p]).wait()

    def save_idx(p):
        for k in range(CH // LANES):
            sl = pl.ds(k * LANES, LANES)
            d2[p][sl] = idv[p][sl]

    def compute(p):
        ap, bp, mp = a[p], b[p], m[p]

        def _crow(r, carry):
            for k in range(H // LANES):
                sl = pl.ds(k * LANES, LANES)
                mp[r, sl] = jnp.maximum(ap[r, sl] + bp[r, sl], 0.0)
            return carry
        lax.fori_loop(0, CH, _crow, 0)

    issue_idx(0, 0)
    issue_idx(1, 1)
    wait_idx(0)
    issue_gather(0)

    def _macro(t, carry):
        @pl.when(t >= 1)
        def _():
            wait_scatter(0)
        wait_idx(1)
        issue_gather(1)
        wait_gather(0)
        save_idx(0)

        @pl.when(t < NCHUNK // 2 - 1)
        def _():
            issue_idx(2 * t + 2, 0)
        compute(0)
        issue_scatter(0)

        @pl.when(t >= 1)
        def _():
            wait_scatter(1)

        @pl.when(t < NCHUNK // 2 - 1)
        def _():
            wait_idx(0)
            issue_gather(0)
        wait_gather(1)
        save_idx(1)

        @pl.when(t < NCHUNK // 2 - 1)
        def _():
            issue_idx(2 * t + 3, 1)
        compute(1)
        issue_scatter(1)
        return carry
    lax.fori_loop(0, NCHUNK // 2, _macro, 0)

    wait_scatter(0)
    wait_scatter(1)
    plsc.subcore_barrier()

    def _wtile(t, carry):
        off = (s + t * NS) * CH
        pltpu.sync_copy(agg_sh.at[pl.ds(off, CH)], m0)
        pltpu.sync_copy(m0, out_hbm.at[c, pl.ds(off, CH)])
        return carry
    lax.fori_loop(0, ntiles, _wtile, 0)


def _edge(ps, pd, src, dst):
    mesh = plsc.VectorSubcoreMesh(core_axis_name="c", subcore_axis_name="s")
    f = pl.kernel(
        _edge_body,
        out_type=jax.ShapeDtypeStruct((NC, NP, H), jnp.float32),
        mesh=mesh,
        scratch_types=[
            pltpu.VMEM((CH, H), jnp.float32),
            pltpu.VMEM((CH, H), jnp.float32),
            pltpu.VMEM((CH, H), jnp.float32),
            pltpu.VMEM((CH, H), jnp.float32),
            pltpu.VMEM((CH, H), jnp.float32),
            pltpu.VMEM((CH, H), jnp.float32),
            pltpu.VMEM((CH,), jnp.int32),
            pltpu.VMEM((CH,), jnp.int32),
            pltpu.VMEM((CH,), jnp.int32),
            pltpu.VMEM((CH,), jnp.int32),
            pltpu.VMEM((CH,), jnp.int32),
            pltpu.VMEM((CH,), jnp.int32),
            pltpu.VMEM_SHARED((NP, H), jnp.float32),
            pltpu.SemaphoreType.DMA,
            pltpu.SemaphoreType.DMA,
            pltpu.SemaphoreType.DMA,
            pltpu.SemaphoreType.DMA,
            pltpu.SemaphoreType.DMA,
            pltpu.SemaphoreType.DMA,
        ],
    )
    return f(ps, pd, src, dst)


def _final_body(s_ref, agg_ref, w_ref, o_ref):
    a = agg_ref[0] + agg_ref[1]
    o_ref[...] = jnp.maximum(
        s_ref[...] + jnp.dot(a, w_ref[...], preferred_element_type=jnp.float32),
        0.0)


def _final(s, agg2, w_agg):
    grid = N // RB
    return pl.pallas_call(
        _final_body,
        grid=(grid,),
        in_specs=[
            pl.BlockSpec((RB, H), lambda i: (i, 0)),
            pl.BlockSpec((NC, RB, H), lambda i: (0, i, 0)),
            pl.BlockSpec((H, H), lambda i: (0, 0)),
        ],
        out_specs=pl.BlockSpec((RB, H), lambda i: (i, 0)),
        out_shape=jax.ShapeDtypeStruct((N, H), jnp.float32),
    )(s, agg2, w_agg)


def kernel(x, edge_index, hidden, W_src, W_dst, W_self, W_agg):
    ei = edge_index.astype(jnp.int32)
    npad = EP - E
    pad_dst = N + (jnp.arange(npad, dtype=jnp.int32) % (NP - N))
    src = jnp.concatenate(
        [ei[0].reshape(NW, E // NW),
         jnp.zeros((NW, npad // NW), jnp.int32)], axis=1).reshape(EP)
    dst = jnp.concatenate(
        [ei[1].reshape(NW, E // NW),
         pad_dst.reshape(NW, npad // NW)], axis=1).reshape(EP)
    wx = jnp.concatenate([W_src[:D], W_dst[:D], W_self[:D]], axis=1)
    wh = jnp.concatenate([W_src[D:], W_dst[D:], W_self[D:]], axis=1)
    ps, pd, s = _proj(x, hidden, wx, wh)
    agg2 = _edge(ps, pd, src, dst)
    return _final(s, agg2, W_agg)


# speedup vs baseline: 11.8619x; 2.0465x over previous
import jax
import jax.numpy as jnp
from jax import lax
from jax.experimental import pallas as pl
from jax.experimental.pallas import tpu as pltpu
from jax.experimental.pallas import tpu_sc as plsc

N = 10000
D = 128
E = 320000
H = 128

NC = 2
NS = 16
NW = NC * NS
CH = 64
NCHUNK = 160
EPW = NCHUNK * CH
EP = NW * EPW
NP = 10048
NT = NP // CH

RB = 1000
LANES = 16


def _proj_body(x_ref, h_ref, wx_ref, wh_ref, ps_ref, pd_ref, s_ref):
    o = (jnp.dot(x_ref[...], wx_ref[...], preferred_element_type=jnp.float32)
         + jnp.dot(h_ref[...], wh_ref[...], preferred_element_type=jnp.float32))
    ps_ref[...] = o[:, :H]
    pd_ref[...] = o[:, H:2 * H]
    s_ref[...] = o[:, 2 * H:]


def _proj(x, hidden, wx, wh):
    grid = N // RB
    return pl.pallas_call(
        _proj_body,
        grid=(grid,),
        in_specs=[
            pl.BlockSpec((RB, D), lambda i: (i, 0)),
            pl.BlockSpec((RB, H), lambda i: (i, 0)),
            pl.BlockSpec((D, 3 * H), lambda i: (0, 0)),
            pl.BlockSpec((H, 3 * H), lambda i: (0, 0)),
        ],
        out_specs=[
            pl.BlockSpec((RB, H), lambda i: (i, 0)),
            pl.BlockSpec((RB, H), lambda i: (i, 0)),
            pl.BlockSpec((RB, H), lambda i: (i, 0)),
        ],
        out_shape=[jax.ShapeDtypeStruct((N, H), jnp.float32)] * 3,
    )(x, hidden, wx, wh)


def _edge_body(ps_hbm, pd_hbm, src_hbm, dst_hbm, out_hbm,
               a0, a1, b0, b1, m0, m1,
               is0, is1, id0, id1, d20, d21, agg_sh,
               gsem0, gsem1, isem0, isem1, ssem0, ssem1):
    c = lax.axis_index("c")
    s = lax.axis_index("s")
    w = c * NS + s
    ebase = w * EPW

    a = (a0, a1)
    b = (b0, b1)
    m = (m0, m1)
    isv = (is0, is1)
    idv = (id0, id1)
    d2 = (d20, d21)
    gsem = (gsem0, gsem1)
    isem = (isem0, isem1)
    ssem = (ssem0, ssem1)

    def _zrow(r, carry):
        for k in range(H // LANES):
            m0[r, pl.ds(k * LANES, LANES)] = jnp.zeros((LANES,), jnp.float32)
        return carry
    lax.fori_loop(0, CH, _zrow, 0)

    ntiles = (NT - s + NS - 1) // NS

    def _ztile(t, carry):
        pltpu.sync_copy(m0, agg_sh.at[pl.ds((s + t * NS) * CH, CH)])
        return carry
    lax.fori_loop(0, ntiles, _ztile, 0)
    plsc.subcore_barrier()

    def issue_idx(j, p):
        pltpu.async_copy(src_hbm.at[pl.ds(ebase + j * CH, CH)], isv[p], isem[p])
        pltpu.async_copy(dst_hbm.at[pl.ds(ebase + j * CH, CH)], idv[p], isem[p])

    def wait_idx(p):
        pltpu.make_async_copy(src_hbm.at[pl.ds(0, CH)], isv[p], isem[p]).wait()
        pltpu.make_async_copy(dst_hbm.at[pl.ds(0, CH)], idv[p], isem[p]).wait()

    def issue_gather(p):
        pltpu.async_copy(ps_hbm.at[isv[p]], a[p], gsem[p])
        pltpu.async_copy(pd_hbm.at[idv[p]], b[p], gsem[p])

    def wait_gather(p):
        pltpu.make_async_copy(ps_hbm.at[isv[p]], a[p], gsem[p]).wait()
        pltpu.make_async_copy(pd_hbm.at[idv[p]], b[p], gsem[p]).wait()

    def issue_scatter(p):
        pltpu.async_copy(m[p], agg_sh.at[d2[p]], ssem[p], add=True)

    def wait_scatter(p):
        pltpu.make_async_copy(m[p], agg_sh.at[d2[p]], ssem[p]).wait()

    def save_idx(p):
        for k in range(CH // LANES):
            sl = pl.ds(k * LANES, LANES)
            d2[p][sl] = idv[p][sl]

    def compute(p):
        ap, bp, mp = a[p], b[p], m[p]

        def _crow(r, carry):
            for k in range(H // LANES):
                sl = pl.ds(k * LANES, LANES)
                mp[r, sl] = jnp.maximum(ap[r, sl] + bp[r, sl], 0.0)
            return carry
        lax.fori_loop(0, CH, _crow, 0)

    issue_idx(0, 0)
    issue_idx(1, 1)
    wait_idx(0)
    issue_gather(0)

    def _macro(t, carry):
        @pl.when(t >= 1)
        def _():
            wait_scatter(0)
        wait_idx(1)
        issue_gather(1)
        wait_gather(0)
        save_idx(0)

        @pl.when(t < NCHUNK // 2 - 1)
        def _():
            issue_idx(2 * t + 2, 0)
        compute(0)
        issue_scatter(0)

        @pl.when(t >= 1)
        def _():
            wait_scatter(1)

        @pl.when(t < NCHUNK // 2 - 1)
        def _():
            wait_idx(0)
            issue_gather(0)
        wait_gather(1)
        save_idx(1)

        @pl.when(t < NCHUNK // 2 - 1)
        def _():
            issue_idx(2 * t + 3, 1)
        compute(1)
        issue_scatter(1)
        return carry
    lax.fori_loop(0, NCHUNK // 2, _macro, 0)

    wait_scatter(0)
    wait_scatter(1)
    plsc.subcore_barrier()

    def _wtile(t, carry):
        off = (s + t * NS) * CH
        pltpu.sync_copy(agg_sh.at[pl.ds(off, CH)], m0)
        pltpu.sync_copy(m0, out_hbm.at[c, pl.ds(off, CH)])
        return carry
    lax.fori_loop(0, ntiles, _wtile, 0)


def _edge(ps, pd, src, dst):
    mesh = plsc.VectorSubcoreMesh(core_axis_name="c", subcore_axis_name="s")
    f = pl.kernel(
        _edge_body,
        out_type=jax.ShapeDtypeStruct((NC, NP, H), jnp.float32),
        mesh=mesh,
        scratch_types=[
            pltpu.VMEM((CH, H), jnp.float32),
            pltpu.VMEM((CH, H), jnp.float32),
            pltpu.VMEM((CH, H), jnp.float32),
            pltpu.VMEM((CH, H), jnp.float32),
            pltpu.VMEM((CH, H), jnp.float32),
            pltpu.VMEM((CH, H), jnp.float32),
            pltpu.VMEM((CH,), jnp.int32),
            pltpu.VMEM((CH,), jnp.int32),
            pltpu.VMEM((CH,), jnp.int32),
            pltpu.VMEM((CH,), jnp.int32),
            pltpu.VMEM((CH,), jnp.int32),
            pltpu.VMEM((CH,), jnp.int32),
            pltpu.VMEM_SHARED((NP, H), jnp.float32),
            pltpu.SemaphoreType.DMA,
            pltpu.SemaphoreType.DMA,
            pltpu.SemaphoreType.DMA,
            pltpu.SemaphoreType.DMA,
            pltpu.SemaphoreType.DMA,
            pltpu.SemaphoreType.DMA,
        ],
    )
    return f(ps, pd, src, dst)


def _final_body(s_ref, agg_ref, w_ref, o_ref):
    a = agg_ref[0] + agg_ref[1]
    o_ref[...] = jnp.maximum(
        s_ref[...] + jnp.dot(a, w_ref[...], preferred_element_type=jnp.float32),
        0.0)


def _final(s, agg2, w_agg):
    grid = N // RB
    return pl.pallas_call(
        _final_body,
        grid=(grid,),
        in_specs=[
            pl.BlockSpec((RB, H), lambda i: (i, 0)),
            pl.BlockSpec((NC, RB, H), lambda i: (0, i, 0)),
            pl.BlockSpec((H, H), lambda i: (0, 0)),
        ],
        out_specs=pl.BlockSpec((RB, H), lambda i: (i, 0)),
        out_shape=jax.ShapeDtypeStruct((N, H), jnp.float32),
    )(s, agg2, w_agg)


def kernel(x, edge_index, hidden, W_src, W_dst, W_self, W_agg):
    ei = edge_index.astype(jnp.int32)
    npad = EP - E
    pad_dst = N + (jnp.arange(npad, dtype=jnp.int32) % (NP - N))
    pad_src = jnp.arange(npad, dtype=jnp.int32) % N
    src = jnp.concatenate(
        [ei[0].reshape(NW, E // NW),
         pad_src.reshape(NW, npad // NW)], axis=1).reshape(EP)
    dst = jnp.concatenate(
        [ei[1].reshape(NW, E // NW),
         pad_dst.reshape(NW, npad // NW)], axis=1).reshape(EP)
    wx = jnp.concatenate([W_src[:D], W_dst[:D], W_self[:D]], axis=1)
    wh = jnp.concatenate([W_src[D:], W_dst[D:], W_self[D:]], axis=1)
    ps, pd, s = _proj(x, hidden, wx, wh)
    agg2 = _edge(ps, pd, src, dst)
    return _final(s, agg2, W_agg)


# speedup vs baseline: 12.6050x; 1.0626x over previous
import jax
import jax.numpy as jnp
from jax import lax
from jax.experimental import pallas as pl
from jax.experimental.pallas import tpu as pltpu
from jax.experimental.pallas import tpu_sc as plsc

N = 10000
D = 128
E = 320000
H = 128

NC = 2
NS = 16
NW = NC * NS
EPW = E // NW
CH = 64
NCHUNK = EPW // CH
TAIL = EPW - NCHUNK * CH
NP = 10048
NT = NP // CH

RB = 1000
LANES = 16


def _proj_body(x_ref, h_ref, wx_ref, wh_ref, ps_ref, pd_ref, s_ref):
    o = (jnp.dot(x_ref[...], wx_ref[...], preferred_element_type=jnp.float32)
         + jnp.dot(h_ref[...], wh_ref[...], preferred_element_type=jnp.float32))
    ps_ref[...] = o[:, :H]
    pd_ref[...] = o[:, H:2 * H]
    s_ref[...] = o[:, 2 * H:]


def _proj(x, hidden, wx, wh):
    grid = N // RB
    return pl.pallas_call(
        _proj_body,
        grid=(grid,),
        in_specs=[
            pl.BlockSpec((RB, D), lambda i: (i, 0)),
            pl.BlockSpec((RB, H), lambda i: (i, 0)),
            pl.BlockSpec((D, 3 * H), lambda i: (0, 0)),
            pl.BlockSpec((H, 3 * H), lambda i: (0, 0)),
        ],
        out_specs=[
            pl.BlockSpec((RB, H), lambda i: (i, 0)),
            pl.BlockSpec((RB, H), lambda i: (i, 0)),
            pl.BlockSpec((RB, H), lambda i: (i, 0)),
        ],
        out_shape=[jax.ShapeDtypeStruct((N, H), jnp.float32)] * 3,
    )(x, hidden, wx, wh)


def _edge_body(ps_hbm, pd_hbm, src_hbm, dst_hbm, out_hbm,
               a0, a1, b0, b1, m0, m1,
               is0, is1, id0, id1, d20, d21, it16, dt16, agg_sh,
               gsem0, gsem1, isem0, isem1, ssem0, ssem1):
    c = lax.axis_index("c")
    s = lax.axis_index("s")
    w = c * NS + s
    ebase = w * EPW

    a = (a0, a1)
    b = (b0, b1)
    m = (m0, m1)
    isv = (is0, is1)
    idv = (id0, id1)
    d2 = (d20, d21)
    gsem = (gsem0, gsem1)
    isem = (isem0, isem1)
    ssem = (ssem0, ssem1)

    def _zrow(r, carry):
        for k in range(H // LANES):
            m0[r, pl.ds(k * LANES, LANES)] = jnp.zeros((LANES,), jnp.float32)
        return carry
    lax.fori_loop(0, CH, _zrow, 0)

    ntiles = (NT - s + NS - 1) // NS

    def _ztile(t, carry):
        pltpu.sync_copy(m0, agg_sh.at[pl.ds((s + t * NS) * CH, CH)])
        return carry
    lax.fori_loop(0, ntiles, _ztile, 0)
    plsc.subcore_barrier()

    pltpu.sync_copy(src_hbm.at[pl.ds(ebase + NCHUNK * CH, TAIL)], it16)
    pltpu.sync_copy(dst_hbm.at[pl.ds(ebase + NCHUNK * CH, TAIL)], dt16)
    pltpu.async_copy(ps_hbm.at[it16], a0.at[pl.ds(0, TAIL)], gsem0).wait()
    pltpu.async_copy(pd_hbm.at[dt16], b0.at[pl.ds(0, TAIL)], gsem0).wait()

    def _trow(r, carry):
        for k in range(H // LANES):
            sl = pl.ds(k * LANES, LANES)
            m0[r, sl] = jnp.maximum(a0[r, sl] + b0[r, sl], 0.0)
        return carry
    lax.fori_loop(0, TAIL, _trow, 0)
    pltpu.sync_copy(m0.at[pl.ds(0, TAIL)], agg_sh.at[dt16], add=True)

    def issue_idx(j, p):
        off = ebase + j * CH
        pltpu.async_copy(src_hbm.at[pl.ds(off, CH)], isv[p], isem[p])
        pltpu.async_copy(dst_hbm.at[pl.ds(off, CH)], idv[p], isem[p])

    def wait_idx(p):
        pltpu.make_async_copy(src_hbm.at[pl.ds(0, CH)], isv[p], isem[p]).wait()
        pltpu.make_async_copy(dst_hbm.at[pl.ds(0, CH)], idv[p], isem[p]).wait()

    def issue_gather(p):
        pltpu.async_copy(ps_hbm.at[isv[p]], a[p], gsem[p])
        pltpu.async_copy(pd_hbm.at[idv[p]], b[p], gsem[p])

    def wait_gather(p):
        pltpu.make_async_copy(ps_hbm.at[isv[p]], a[p], gsem[p]).wait()
        pltpu.make_async_copy(pd_hbm.at[idv[p]], b[p], gsem[p]).wait()

    def issue_scatter(p):
        pltpu.async_copy(m[p], agg_sh.at[d2[p]], ssem[p], add=True)

    def wait_scatter(p):
        pltpu.make_async_copy(m[p], agg_sh.at[d2[p]], ssem[p]).wait()

    def save_idx(p):
        for k in range(CH // LANES):
            sl = pl.ds(k * LANES, LANES)
            d2[p][sl] = idv[p][sl]

    def compute(p):
        ap, bp, mp = a[p], b[p], m[p]

        def _crow(r, carry):
            for k in range(H // LANES):
                sl = pl.ds(k * LANES, LANES)
                mp[r, sl] = jnp.maximum(ap[r, sl] + bp[r, sl], 0.0)
            return carry
        lax.fori_loop(0, CH, _crow, 0)

    issue_idx(0, 0)
    issue_idx(1, 1)
    wait_idx(0)
    issue_gather(0)

    def _macro(t, carry):
        @pl.when(t >= 1)
        def _():
            wait_scatter(0)
        wait_idx(1)
        issue_gather(1)
        wait_gather(0)
        save_idx(0)

        @pl.when(t < NCHUNK // 2 - 1)
        def _():
            issue_idx(2 * t + 2, 0)
        compute(0)
        issue_scatter(0)

        @pl.when(t >= 1)
        def _():
            wait_scatter(1)

        @pl.when(t < NCHUNK // 2 - 1)
        def _():
            wait_idx(0)
            issue_gather(0)
        wait_gather(1)
        save_idx(1)

        @pl.when(t < NCHUNK // 2 - 1)
        def _():
            issue_idx(2 * t + 3, 1)
        compute(1)
        issue_scatter(1)
        return carry
    lax.fori_loop(0, NCHUNK // 2, _macro, 0)

    wait_scatter(0)
    wait_scatter(1)
    plsc.subcore_barrier()

    def _wtile(t, carry):
        off = (s + t * NS) * CH
        pltpu.sync_copy(agg_sh.at[pl.ds(off, CH)], m0)
        pltpu.sync_copy(m0, out_hbm.at[c, pl.ds(off, CH)])
        return carry
    lax.fori_loop(0, ntiles, _wtile, 0)


def _edge(ps, pd, src, dst):
    mesh = plsc.VectorSubcoreMesh(core_axis_name="c", subcore_axis_name="s")
    f = pl.kernel(
        _edge_body,
        out_type=jax.ShapeDtypeStruct((NC, NP, H), jnp.float32),
        mesh=mesh,
        scratch_types=[
            pltpu.VMEM((CH, H), jnp.float32),
            pltpu.VMEM((CH, H), jnp.float32),
            pltpu.VMEM((CH, H), jnp.float32),
            pltpu.VMEM((CH, H), jnp.float32),
            pltpu.VMEM((CH, H), jnp.float32),
            pltpu.VMEM((CH, H), jnp.float32),
            pltpu.VMEM((CH,), jnp.int32),
            pltpu.VMEM((CH,), jnp.int32),
            pltpu.VMEM((CH,), jnp.int32),
            pltpu.VMEM((CH,), jnp.int32),
            pltpu.VMEM((CH,), jnp.int32),
            pltpu.VMEM((CH,), jnp.int32),
            pltpu.VMEM((TAIL,), jnp.int32),
            pltpu.VMEM((TAIL,), jnp.int32),
            pltpu.VMEM_SHARED((NP, H), jnp.float32),
            pltpu.SemaphoreType.DMA,
            pltpu.SemaphoreType.DMA,
            pltpu.SemaphoreType.DMA,
            pltpu.SemaphoreType.DMA,
            pltpu.SemaphoreType.DMA,
            pltpu.SemaphoreType.DMA,
        ],
    )
    return f(ps, pd, src, dst)


def _final_body(s_ref, agg_ref, w_ref, o_ref):
    a = agg_ref[0] + agg_ref[1]
    o_ref[...] = jnp.maximum(
        s_ref[...] + jnp.dot(a, w_ref[...], preferred_element_type=jnp.float32),
        0.0)


def _final(s, agg2, w_agg):
    grid = N // RB
    return pl.pallas_call(
        _final_body,
        grid=(grid,),
        in_specs=[
            pl.BlockSpec((RB, H), lambda i: (i, 0)),
            pl.BlockSpec((NC, RB, H), lambda i: (0, i, 0)),
            pl.BlockSpec((H, H), lambda i: (0, 0)),
        ],
        out_specs=pl.BlockSpec((RB, H), lambda i: (i, 0)),
        out_shape=jax.ShapeDtypeStruct((N, H), jnp.float32),
    )(s, agg2, w_agg)


def kernel(x, edge_index, hidden, W_src, W_dst, W_self, W_agg):
    ei = edge_index.astype(jnp.int32)
    src, dst = ei[0], ei[1]
    wx = jnp.concatenate([W_src[:D], W_dst[:D], W_self[:D]], axis=1)
    wh = jnp.concatenate([W_src[D:], W_dst[D:], W_self[D:]], axis=1)
    ps, pd, s = _proj(x, hidden, wx, wh)
    agg2 = _edge(ps, pd, src, dst)
    return _final(s, agg2, W_agg)
